# baseline (speedup 1.0000x reference)
import numpy as np

import jax
import jax.numpy as jnp
from jax.experimental import pallas as pl
from jax.experimental.pallas import tpu as pltpu

B = 4096
S = 12
D = 1024
K = 3

BB = 128


def _threefry2x32(k1, k2, x1, x2):
    rot = ((13, 15, 26, 6), (17, 29, 16, 24))
    ks = (k1, k2, np.uint32(k1 ^ k2 ^ np.uint32(0x1BD11BDA)))
    x = [x1 + ks[0], x2 + ks[1]]
    for i in range(5):
        for r in rot[i % 2]:
            x[0] = x[0] + x[1]
            x[1] = (x[1] << np.uint32(r)) | (x[1] >> np.uint32(32 - r))
            x[1] = x[0] ^ x[1]
        x[0] = x[0] + ks[(i + 1) % 3]
        x[1] = x[1] + ks[(i + 2) % 3] + np.uint32(i + 1)
    return x[0], x[1]


def _uniform_np(seed: int, shape) -> np.ndarray:
    k1, k2 = np.uint32(seed >> 32), np.uint32(seed & 0xFFFFFFFF)
    n = int(np.prod(shape))
    flat = np.arange(n, dtype=np.uint64)
    c1 = (flat >> np.uint64(32)).astype(np.uint32)
    c2 = (flat & np.uint64(0xFFFFFFFF)).astype(np.uint32)
    b1, b2 = _threefry2x32(k1, k2, c1, c2)
    bits = b1 ^ b2
    fb = (bits >> np.uint32(9)) | np.uint32(0x3F800000)
    return (fb.view(np.float32) - np.float32(1.0)).reshape(shape)


def _selection() -> np.ndarray:
    rand = _uniform_np(1, (B, S))
    return np.argsort(rand, axis=-1, kind="stable")[:, :K].astype(np.int32)


_SEL = _selection()
_OH = np.zeros((B, K, S), dtype=np.float32)
_OH[np.arange(B)[:, None], np.arange(K)[None, :], _SEL] = 1.0


def _body(in_ref, oh_ref, out_ref):
    for bl in range(BB):
        out_ref[bl] = jnp.dot(
            oh_ref[bl], in_ref[bl], preferred_element_type=jnp.float32,
            precision=jax.lax.Precision.HIGHEST
        )


def kernel(image_latent):
    return pl.pallas_call(
        _body,
        grid=(B // BB,),
        in_specs=[
            pl.BlockSpec((BB, S, D), lambda g: (g, 0, 0)),
            pl.BlockSpec((BB, K, S), lambda g: (g, 0, 0)),
        ],
        out_specs=pl.BlockSpec((BB, K, D), lambda g: (g, 0, 0)),
        out_shape=jax.ShapeDtypeStruct((B, K, D), jnp.float32),
        compiler_params=pltpu.CompilerParams(
            dimension_semantics=("arbitrary",),
        ),
    )(image_latent, jnp.asarray(_OH))


# speedup vs baseline: 1.6339x; 1.6339x over previous
import functools

import numpy as np

import jax
import jax.numpy as jnp
from jax import lax
from jax.experimental import pallas as pl
from jax.experimental.pallas import tpu as pltpu
from jax.experimental.pallas import tpu_sc as plsc

B = 4096
S = 12
D = 1024
K = 3

NC = 2
NS = 16
NW = NC * NS

BPW = B // NW
CB = 8
NCHUNK = BPW // CB
NPAIR = CB * K
TBL_COLS = 128


def _threefry2x32(k1, k2, x1, x2):
    rot = ((13, 15, 26, 6), (17, 29, 16, 24))
    ks = (k1, k2, np.uint32(k1 ^ k2 ^ np.uint32(0x1BD11BDA)))
    x = [x1 + ks[0], x2 + ks[1]]
    for i in range(5):
        for r in rot[i % 2]:
            x[0] = x[0] + x[1]
            x[1] = (x[1] << np.uint32(r)) | (x[1] >> np.uint32(32 - r))
            x[1] = x[0] ^ x[1]
        x[0] = x[0] + ks[(i + 1) % 3]
        x[1] = x[1] + ks[(i + 2) % 3] + np.uint32(i + 1)
    return x[0], x[1]


def _uniform_np(seed: int, shape) -> np.ndarray:
    k1, k2 = np.uint32(seed >> 32), np.uint32(seed & 0xFFFFFFFF)
    n = int(np.prod(shape))
    flat = np.arange(n, dtype=np.uint64)
    c1 = (flat >> np.uint64(32)).astype(np.uint32)
    c2 = (flat & np.uint64(0xFFFFFFFF)).astype(np.uint32)
    b1, b2 = _threefry2x32(k1, k2, c1, c2)
    bits = b1 ^ b2
    fb = (bits >> np.uint32(9)) | np.uint32(0x3F800000)
    return (fb.view(np.float32) - np.float32(1.0)).reshape(shape)


def _selection() -> np.ndarray:
    rand = _uniform_np(1, (B, S))
    return np.argsort(rand, axis=-1, kind="stable")[:, :K].astype(np.int32)


def _tables() -> np.ndarray:
    sel = _selection()
    tbl = np.zeros((NW, NCHUNK, TBL_COLS), dtype=np.int32)
    for w in range(NW):
        for c in range(NCHUNK):
            b0 = w * BPW + c * CB
            tbl[w, c, :NPAIR] = sel[b0 : b0 + CB].reshape(-1)
    return tbl


def _build_sc_kernel():
    mesh = plsc.VectorSubcoreMesh(core_axis_name="c", subcore_axis_name="s")
    scratch = [
        pltpu.VMEM((NCHUNK, TBL_COLS), jnp.int32),
        pltpu.VMEM((CB, K, D), jnp.float32),
        pltpu.VMEM((CB, K, D), jnp.float32),
        pltpu.SemaphoreType.DMA,
        pltpu.SemaphoreType.DMA,
        pltpu.SemaphoreType.DMA,
        pltpu.SemaphoreType.DMA,
    ]

    @functools.partial(
        pl.kernel,
        mesh=mesh,
        out_type=jax.ShapeDtypeStruct((B, K, D), jnp.float32),
        scratch_types=scratch,
        compiler_params=pltpu.CompilerParams(
            needs_layout_passes=False,
            disable_bounds_checks=True,
            disable_semaphore_checks=True,
            skip_device_barrier=True,
        ),
    )
    def body(img, tbl, out, tbl_v, slab0, slab1, gsem0, gsem1, wsem0, wsem1):
        wid = lax.axis_index("s") * NC + lax.axis_index("c")
        pltpu.sync_copy(tbl.at[wid], tbl_v)
        lanes = lax.iota(jnp.int32, 16)

        def drain_write(slab, wsem):
            pltpu.make_async_copy(slab, out.at[pl.ds(0, CB)], wsem).wait()

        def do_chunk(c, slab, gsem, wsem):
            b0 = wid * BPW + c * CB
            svec0 = tbl_v[c, 0:16]
            svec1 = tbl_v[c, 16:32]
            for p in range(NPAIR):
                svec = svec0 if p < 16 else svec1
                lane = p % 16
                sval = lax.reduce_max(
                    jnp.where(lanes == lane, svec, jnp.int32(0)), axes=(0,)
                )
                bl, k = divmod(p, K)
                pltpu.async_copy(
                    img.at[pl.ds(b0 + bl, 1), pl.ds(sval, 1)],
                    slab.at[pl.ds(bl, 1), pl.ds(k, 1)],
                    gsem,
                )
            pltpu.make_async_copy(
                img.at[pl.ds(0, CB), pl.ds(0, K)], slab, gsem
            ).wait()
            pltpu.async_copy(slab, out.at[pl.ds(b0, CB)], wsem)

        def loop_body(g, carry):
            @pl.when(g > 0)
            def _():
                drain_write(slab0, wsem0)

            do_chunk(2 * g, slab0, gsem0, wsem0)

            @pl.when(g > 0)
            def _():
                drain_write(slab1, wsem1)

            do_chunk(2 * g + 1, slab1, gsem1, wsem1)
            return carry

        lax.fori_loop(0, NCHUNK // 2, loop_body, jnp.int32(0))
        drain_write(slab0, wsem0)
        drain_write(slab1, wsem1)

    return body


_TBL = _tables()
_SC_KERNEL = _build_sc_kernel()


def kernel(image_latent):
    return _SC_KERNEL(image_latent, jnp.asarray(_TBL))
